# Initial kernel scaffold; baseline (speedup 1.0000x reference)
import jax
import jax.numpy as jnp
from jax.experimental import pallas as pl


def kernel(atomic_numbers, zs):
    raise NotImplementedError("write your pallas kernel here")


# speedup vs baseline: 11.8474x; 11.8474x over previous
import functools

import jax
import jax.numpy as jnp
from jax import lax
from jax.experimental import pallas as pl
from jax.experimental.pallas import tpu as pltpu
from jax.experimental.pallas import tpu_sc as plsc

_NC = 2
_NS = 16
_NW = _NC * _NS
_L = 16
_TBL = 64


def _pick_chunk(n):
    for ch in (2000, 1600, 1024, 512, 400, 200, 80, 16):
        if n % ch == 0:
            return ch
    raise ValueError(f"n={n} not divisible by any supported chunk size")


def _make_sc_encode(n, C):
    CH = _pick_chunk(n)
    CHW = CH * C
    NCHUNK = n // CH
    MAXC = -(-NCHUNK // _NW)
    G = CH // _L

    mesh = plsc.VectorSubcoreMesh(
        core_axis_name="c", subcore_axis_name="s",
        num_cores=_NC, num_subcores=_NS)

    @functools.partial(
        pl.kernel,
        out_type=jax.ShapeDtypeStruct((n * C,), jnp.float32),
        mesh=mesh,
        compiler_params=pltpu.CompilerParams(needs_layout_passes=False),
        scratch_types=[
            pltpu.VMEM((_L,), jnp.int32),
            pltpu.VMEM((_TBL,), jnp.int32),
            pltpu.VMEM((_TBL,), jnp.int32),
            pltpu.VMEM((_TBL,), jnp.float32),
            pltpu.VMEM((_TBL,), jnp.float32),
            pltpu.VMEM((CH,), jnp.int32),
            pltpu.VMEM((CHW,), jnp.float32),
            pltpu.VMEM((CHW,), jnp.float32),
            pltpu.SemaphoreType.DMA,
            pltpu.SemaphoreType.DMA,
        ],
    )
    def encode(az_hbm, zs_hbm, out_hbm, zs_v, clo_t, chi_t, vlo_t, vhi_t,
               idx_v, outb0, outb1, sem0, sem1):
        wid = lax.axis_index("s") * _NC + lax.axis_index("c")
        outbs = (outb0, outb1)
        sems = (sem0, sem1)

        pltpu.sync_copy(zs_hbm, zs_v.at[pl.ds(0, C)])

        iota = lax.iota(jnp.int32, _L)
        zrow = zs_v[...]
        zk = [zrow[k] for k in range(C)]
        for t in range(_TBL // _L):
            v = iota + (t * _L)
            j = jnp.zeros((_L,), jnp.int32)
            for k in range(C):
                j = j + jnp.where(zk[k] < v, 1, 0).astype(jnp.int32)
            jc = jnp.minimum(j, C - 1)
            lo = jnp.maximum(jc - 1, 0)
            zhi = plsc.load_gather(zs_v, [jc])
            zlo = plsc.load_gather(zs_v, [lo])
            exact = zhi == v
            v_f = v.astype(jnp.float32)
            zhi_f = zhi.astype(jnp.float32)
            zlo_f = zlo.astype(jnp.float32)
            denom = jnp.maximum(zhi_f - zlo_f, 1.0)
            w_lo = (zhi_f - v_f) / denom
            w_hi = (v_f - zlo_f) / denom
            sl = pl.ds(t * _L, _L)
            clo_t[sl] = jnp.where(exact, jc, lo)
            chi_t[sl] = jc
            vlo_t[sl] = jnp.where(exact, 1.0, w_lo)
            vhi_t[sl] = jnp.where(exact, 1.0, w_hi)

        zeros = jnp.zeros((_L,), jnp.float32)
        iotaC = iota * C

        def do_group(g, outb):
            z = idx_v[pl.ds(g * _L, _L)]
            z = jnp.minimum(jnp.maximum(z, 0), _TBL - 1)
            clo = plsc.load_gather(clo_t, [z])
            chi = plsc.load_gather(chi_t, [z])
            vlo = plsc.load_gather(vlo_t, [z])
            vhi = plsc.load_gather(vhi_t, [z])
            base = g * (_L * C)
            for r in range(C):
                outb[pl.ds(base + r * _L, _L)] = zeros
            fl = base + iotaC
            plsc.store_scatter(outb, [fl + clo], vlo)
            plsc.store_scatter(outb, [fl + chi], vhi)
            return 0

        for c in range(MAXC):
            b = c % 2
            outb = outbs[b]
            cid = c * _NW + wid

            @pl.when(cid < NCHUNK)
            def _():
                if c >= 2:
                    pltpu.make_async_copy(
                        out_hbm.at[pl.ds(0, CHW)], outb, sems[b]).wait()
                pltpu.sync_copy(az_hbm.at[pl.ds(cid * CH, CH)], idx_v)
                lax.fori_loop(0, G, lambda g, _: do_group(g, outb), 0)
                pltpu.async_copy(
                    outb, out_hbm.at[pl.ds(cid * CHW, CHW)], sems[b])

        for b in range(2):
            pltpu.make_async_copy(
                out_hbm.at[pl.ds(0, CHW)], outbs[b], sems[b]).wait()

    return encode


def kernel(atomic_numbers, zs):
    n = atomic_numbers.shape[0]
    C = zs.shape[0]
    enc = _make_sc_encode(n, C)
    out_flat = enc(atomic_numbers.astype(jnp.int32), zs.astype(jnp.int32))
    return out_flat.reshape(n, C)


# speedup vs baseline: 17.1376x; 1.4465x over previous
import functools

import jax
import jax.numpy as jnp
from jax import lax
from jax.experimental import pallas as pl
from jax.experimental.pallas import tpu as pltpu
from jax.experimental.pallas import tpu_sc as plsc

_NC = 2
_NS = 16
_NW = _NC * _NS
_L = 16
_TBL = 64


def _pick_chunk(n):
    for ch in (2000, 1600, 1024, 512, 400, 200, 80, 16):
        if n % ch == 0:
            return ch
    raise ValueError(f"n={n} not divisible by any supported chunk size")


def _make_sc_encode(n, C):
    CH = _pick_chunk(n)
    NCHUNK = n // CH
    MAXC = -(-NCHUNK // _NW)
    G = CH // _L

    mesh = plsc.VectorSubcoreMesh(
        core_axis_name="c", subcore_axis_name="s",
        num_cores=_NC, num_subcores=_NS)

    @functools.partial(
        pl.kernel,
        out_type=jax.ShapeDtypeStruct((n, C), jnp.float32),
        mesh=mesh,
        compiler_params=pltpu.CompilerParams(
            needs_layout_passes=False, use_tc_tiling_on_sc=False),
        scratch_types=[
            pltpu.VMEM((_L,), jnp.int32),
            pltpu.VMEM((_TBL,), jnp.int32),
            pltpu.VMEM((_TBL,), jnp.int32),
            pltpu.VMEM((_TBL,), jnp.float32),
            pltpu.VMEM((_TBL,), jnp.float32),
            pltpu.VMEM((CH,), jnp.int32),
            pltpu.VMEM((CH, C), jnp.float32),
            pltpu.VMEM((CH, C), jnp.float32),
            pltpu.SemaphoreType.DMA,
            pltpu.SemaphoreType.DMA,
        ],
    )
    def encode(az_hbm, zs_hbm, out_hbm, zs_v, clo_t, chi_t, vlo_t, vhi_t,
               idx_v, outb0, outb1, sem0, sem1):
        wid = lax.axis_index("s") * _NC + lax.axis_index("c")
        outbs = (outb0, outb1)
        sems = (sem0, sem1)

        pltpu.sync_copy(zs_hbm, zs_v.at[pl.ds(0, C)])

        iota = lax.iota(jnp.int32, _L)
        zrow = zs_v[...]
        zk = [zrow[k] for k in range(C)]
        for t in range(_TBL // _L):
            v = iota + (t * _L)
            j = jnp.zeros((_L,), jnp.int32)
            for k in range(C):
                j = j + jnp.where(zk[k] < v, 1, 0).astype(jnp.int32)
            jc = jnp.minimum(j, C - 1)
            lo = jnp.maximum(jc - 1, 0)
            zhi = plsc.load_gather(zs_v, [jc])
            zlo = plsc.load_gather(zs_v, [lo])
            exact = zhi == v
            v_f = v.astype(jnp.float32)
            zhi_f = zhi.astype(jnp.float32)
            zlo_f = zlo.astype(jnp.float32)
            denom = jnp.maximum(zhi_f - zlo_f, 1.0)
            w_lo = (zhi_f - v_f) / denom
            w_hi = (v_f - zlo_f) / denom
            sl = pl.ds(t * _L, _L)
            clo_t[sl] = jnp.where(exact, jc, lo)
            chi_t[sl] = jc
            vlo_t[sl] = jnp.where(exact, 1.0, w_lo)
            vhi_t[sl] = jnp.where(exact, 1.0, w_hi)

        zeros = jnp.zeros((_L,), jnp.float32)

        def do_group(g, outb):
            z = idx_v[pl.ds(g * _L, _L)]
            z = jnp.minimum(jnp.maximum(z, 0), _TBL - 1)
            clo = plsc.load_gather(clo_t, [z])
            chi = plsc.load_gather(chi_t, [z])
            vlo = plsc.load_gather(vlo_t, [z])
            vhi = plsc.load_gather(vhi_t, [z])
            rows = g * _L + iota
            czero = clo * 0
            for r in range(C):
                plsc.store_scatter(outb, [rows, czero + r], zeros)
            plsc.store_scatter(outb, [rows, clo], vlo)
            plsc.store_scatter(outb, [rows, chi], vhi)
            return 0

        for c in range(MAXC):
            b = c % 2
            outb = outbs[b]
            cid = c * _NW + wid

            @pl.when(cid < NCHUNK)
            def _():
                if c >= 2:
                    pltpu.make_async_copy(
                        out_hbm.at[pl.ds(0, CH)], outb, sems[b]).wait()
                pltpu.sync_copy(az_hbm.at[pl.ds(cid * CH, CH)], idx_v)
                lax.fori_loop(0, G, lambda g, _: do_group(g, outb), 0)
                pltpu.async_copy(
                    outb, out_hbm.at[pl.ds(cid * CH, CH)], sems[b])

        for b in range(2):
            pltpu.make_async_copy(
                out_hbm.at[pl.ds(0, CH)], outbs[b], sems[b]).wait()

    return encode


def kernel(atomic_numbers, zs):
    n = atomic_numbers.shape[0]
    C = zs.shape[0]
    enc = _make_sc_encode(n, C)
    return enc(atomic_numbers.astype(jnp.int32), zs.astype(jnp.int32))


# speedup vs baseline: 118.6706x; 6.9246x over previous
import functools

import jax
import jax.numpy as jnp
from jax import lax
from jax.experimental import pallas as pl
from jax.experimental.pallas import tpu as pltpu
from jax.experimental.pallas import tpu_sc as plsc

_NC = 2
_NS = 16
_NW = _NC * _NS
_L = 16
_TBL = 64
_CHT = 16


def _make_sc_encode(n, C):
    TRC = -(-C // 8)
    NT = -(-n // 128)
    CH = _CHT * 128
    NCHUNK = -(-NT // _CHT)
    MAXC = -(-NCHUNK // _NW)
    TILE_W = _CHT * 1024

    LAST_T0 = NT - _CHT
    LAST_R0 = LAST_T0 * 128
    LAST_G = (n - LAST_R0) // _L
    LAST_SRC = n - CH
    assert LAST_R0 - LAST_SRC >= 0 and (n - LAST_R0) % _L == 0
    assert LAST_SRC % 8 == 0

    mesh = plsc.VectorSubcoreMesh(
        core_axis_name="c", subcore_axis_name="s",
        num_cores=_NC, num_subcores=_NS)

    @functools.partial(
        pl.kernel,
        out_type=jax.ShapeDtypeStruct((TRC, NT, 8, 128), jnp.float32),
        mesh=mesh,
        compiler_params=pltpu.CompilerParams(
            needs_layout_passes=False, use_tc_tiling_on_sc=False),
        scratch_types=[
            pltpu.VMEM((_L,), jnp.int32),
            pltpu.VMEM((_TBL,), jnp.int32),
            pltpu.VMEM((_TBL,), jnp.int32),
            pltpu.VMEM((_TBL,), jnp.float32),
            pltpu.VMEM((_TBL,), jnp.float32),
            pltpu.VMEM((CH,), jnp.int32),
            pltpu.VMEM((TRC, _CHT, 8, 128), jnp.float32),
            pltpu.VMEM((TRC, _CHT, 8, 128), jnp.float32),
            pltpu.SemaphoreType.DMA,
            pltpu.SemaphoreType.DMA,
            pltpu.SemaphoreType.DMA,
            pltpu.SemaphoreType.DMA,
        ],
    )
    def encode(az_hbm, zs_hbm, out_hbm, zs_v, clo_t, chi_t, vlo_t, vhi_t,
               idx_v, outb0, outb1, sem00, sem01, sem10, sem11):
        wid = lax.axis_index("s") * _NC + lax.axis_index("c")
        outbs = (outb0, outb1)
        sems = ((sem00, sem01), (sem10, sem11))

        pltpu.sync_copy(zs_hbm, zs_v.at[pl.ds(0, C)])

        iota = lax.iota(jnp.int32, _L)
        zrow = zs_v[...]
        zk = [zrow[k] for k in range(C)]
        for t in range(_TBL // _L):
            v = iota + (t * _L)
            j = jnp.zeros((_L,), jnp.int32)
            for k in range(C):
                j = j + jnp.where(zk[k] < v, 1, 0).astype(jnp.int32)
            jc = jnp.minimum(j, C - 1)
            lo = jnp.maximum(jc - 1, 0)
            zhi = plsc.load_gather(zs_v, [jc])
            zlo = plsc.load_gather(zs_v, [lo])
            exact = zhi == v
            v_f = v.astype(jnp.float32)
            zhi_f = zhi.astype(jnp.float32)
            zlo_f = zlo.astype(jnp.float32)
            denom = jnp.maximum(zhi_f - zlo_f, 1.0)
            w_lo = (zhi_f - v_f) / denom
            w_hi = (v_f - zlo_f) / denom
            sl = pl.ds(t * _L, _L)
            clo_t[sl] = jnp.where(exact, jc, lo)
            chi_t[sl] = jc
            vlo_t[sl] = jnp.where(exact, 1.0, w_lo)
            vhi_t[sl] = jnp.where(exact, 1.0, w_hi)

        zeros = jnp.zeros((_L,), jnp.float32)

        def do_group(g, goff, outb):
            z = idx_v[pl.ds((g + goff) * _L, _L)]
            z = jnp.minimum(jnp.maximum(z, 0), _TBL - 1)
            clo = plsc.load_gather(clo_t, [z])
            chi = plsc.load_gather(chi_t, [z])
            vlo = plsc.load_gather(vlo_t, [z])
            vhi = plsc.load_gather(vhi_t, [z])
            rloc = g * _L
            t = rloc // 128
            lst = rloc % 128
            for cc in range(C):
                outb[cc // 8, t, cc % 8, pl.ds(lst, _L)] = zeros
            lvec = lst + iota
            tvec = jnp.broadcast_to(t, (_L,))
            plsc.store_scatter(
                outb, [clo >> 3, tvec, clo & 7, lvec], vlo)
            plsc.store_scatter(
                outb, [chi >> 3, tvec, chi & 7, lvec], vhi)
            return 0

        for c in range(MAXC):
            b = c % 2
            outb = outbs[b]
            cid = c * _NW + wid

            @pl.when(cid < NCHUNK)
            def _():
                last = cid == NCHUNK - 1
                src = jnp.where(last, LAST_SRC, cid * CH)
                goff = jnp.where(last, (LAST_R0 - LAST_SRC) // _L, 0)
                ngrp = jnp.where(last, LAST_G, CH // _L)
                t0 = jnp.where(last, LAST_T0, cid * _CHT)
                if c >= 2:
                    for tr in range(TRC):
                        pltpu.make_async_copy(
                            out_hbm.at[tr, pl.ds(0, _CHT)], outb.at[tr],
                            sems[b][tr]).wait()
                pltpu.sync_copy(az_hbm.at[pl.ds(src, CH)], idx_v)
                lax.fori_loop(
                    0, ngrp, lambda g, _: do_group(g, goff, outb), 0)
                for tr in range(TRC):
                    pltpu.async_copy(
                        outb.at[tr], out_hbm.at[tr, pl.ds(t0, _CHT)],
                        sems[b][tr])

        for b in range(2):
            for tr in range(TRC):
                pltpu.make_async_copy(
                    out_hbm.at[tr, pl.ds(0, _CHT)], outbs[b].at[tr],
                    sems[b][tr]).wait()

    return encode


def kernel(atomic_numbers, zs):
    n = atomic_numbers.shape[0]
    C = zs.shape[0]
    TRC = -(-C // 8)
    NT = -(-n // 128)
    enc = _make_sc_encode(n, C)
    out4 = enc(atomic_numbers.astype(jnp.int32), zs.astype(jnp.int32))
    return out4.transpose(1, 3, 0, 2).reshape(NT * 128, TRC * 8)[:n, :C]


# speedup vs baseline: 185.6904x; 1.5648x over previous
import functools

import jax
import jax.numpy as jnp
from jax import lax
from jax.experimental import pallas as pl
from jax.experimental.pallas import tpu as pltpu
from jax.experimental.pallas import tpu_sc as plsc

_NC = 2
_NS = 16
_NW = _NC * _NS
_L = 16
_TBL = 64
_CHT = 16


def _make_sc_encode(n, C):
    TRC = -(-C // 8)
    NT = -(-n // 128)
    CH = _CHT * 128
    NCHUNK = -(-NT // _CHT)
    MAXC = -(-NCHUNK // _NW)

    LAST_T0 = NT - _CHT
    LAST_R0 = LAST_T0 * 128
    LAST_G = (n - LAST_R0) // _L
    LAST_SRC = n - CH
    assert LAST_R0 - LAST_SRC >= 0 and (n - LAST_R0) % _L == 0
    assert LAST_SRC % 8 == 0

    mesh = plsc.VectorSubcoreMesh(
        core_axis_name="c", subcore_axis_name="s",
        num_cores=_NC, num_subcores=_NS)

    @functools.partial(
        pl.kernel,
        out_type=jax.ShapeDtypeStruct((TRC, NT, 8, 128), jnp.float32),
        mesh=mesh,
        compiler_params=pltpu.CompilerParams(
            needs_layout_passes=False, use_tc_tiling_on_sc=False),
        scratch_types=[
            pltpu.VMEM((_L,), jnp.int32),
            pltpu.VMEM((_TBL,), jnp.int32),
            pltpu.VMEM((_TBL,), jnp.float32),
            pltpu.VMEM((_TBL,), jnp.float32),
            pltpu.VMEM((CH,), jnp.int32),
            pltpu.VMEM((CH,), jnp.int32),
            pltpu.VMEM((TRC, _CHT, 8, 128), jnp.float32),
            pltpu.VMEM((TRC, _CHT, 8, 128), jnp.float32),
            pltpu.SemaphoreType.DMA,
            pltpu.SemaphoreType.DMA,
            pltpu.SemaphoreType.DMA,
            pltpu.SemaphoreType.DMA,
            pltpu.SemaphoreType.DMA,
            pltpu.SemaphoreType.DMA,
        ],
    )
    def encode(az_hbm, zs_hbm, out_hbm, zs_v, cpk_t, vlo_t, vhi_t,
               idx0, idx1, outb0, outb1, sem00, sem01, sem10, sem11,
               isem0, isem1):
        wid = lax.axis_index("s") * _NC + lax.axis_index("c")
        outbs = (outb0, outb1)
        idxbufs = (idx0, idx1)
        isems = (isem0, isem1)
        sems = ((sem00, sem01), (sem10, sem11))

        pltpu.sync_copy(zs_hbm, zs_v.at[pl.ds(0, C)])

        iota = lax.iota(jnp.int32, _L)
        zrow = zs_v[...]
        zk = [zrow[k] for k in range(C)]
        for t in range(_TBL // _L):
            v = iota + (t * _L)
            j = jnp.zeros((_L,), jnp.int32)
            for k in range(C):
                j = j + jnp.where(zk[k] < v, 1, 0).astype(jnp.int32)
            jc = jnp.minimum(j, C - 1)
            lo = jnp.maximum(jc - 1, 0)
            zhi = plsc.load_gather(zs_v, [jc])
            zlo = plsc.load_gather(zs_v, [lo])
            exact = zhi == v
            v_f = v.astype(jnp.float32)
            zhi_f = zhi.astype(jnp.float32)
            zlo_f = zlo.astype(jnp.float32)
            denom = jnp.maximum(zhi_f - zlo_f, 1.0)
            w_lo = (zhi_f - v_f) / denom
            w_hi = (v_f - zlo_f) / denom
            sl = pl.ds(t * _L, _L)
            cpk_t[sl] = jnp.where(exact, jc, lo) + jc * 256
            vlo_t[sl] = jnp.where(exact, 1.0, w_lo)
            vhi_t[sl] = jnp.where(exact, 1.0, w_hi)

        zeros = jnp.zeros((_L,), jnp.float32)

        def issue_idx(c):
            cid = c * _NW + wid

            @pl.when(cid < NCHUNK)
            def _():
                src = jnp.where(cid == NCHUNK - 1, LAST_SRC, cid * CH)
                pltpu.async_copy(
                    az_hbm.at[pl.ds(src, CH)], idxbufs[c % 2], isems[c % 2])

        issue_idx(0)
        for c in range(MAXC):
            if c + 1 < MAXC:
                issue_idx(c + 1)
            b = c % 2
            outb = outbs[b]
            idx_v = idxbufs[b]
            cid = c * _NW + wid

            @pl.when(cid < NCHUNK)
            def _():
                last = cid == NCHUNK - 1
                goff = jnp.where(last, (LAST_R0 - LAST_SRC) // _L, 0)
                ngrp = jnp.where(last, LAST_G, CH // _L)
                t0 = jnp.where(last, LAST_T0, cid * _CHT)
                if c >= 2:
                    for tr in range(TRC):
                        pltpu.make_async_copy(
                            out_hbm.at[tr, pl.ds(0, _CHT)], outb.at[tr],
                            sems[b][tr]).wait()
                pltpu.make_async_copy(
                    az_hbm.at[pl.ds(0, CH)], idx_v, isems[b]).wait()

                @plsc.parallel_loop(0, ngrp, unroll=2)
                def _(g):
                    z = idx_v[pl.ds((g + goff) * _L, _L)]
                    z = jnp.minimum(jnp.maximum(z, 0), _TBL - 1)
                    cpk = plsc.load_gather(cpk_t, [z])
                    vlo = plsc.load_gather(vlo_t, [z])
                    vhi = plsc.load_gather(vhi_t, [z])
                    clo = cpk & 255
                    chi = cpk >> 8
                    rloc = g * _L
                    t = rloc // 128
                    lst = rloc % 128
                    for cc in range(C):
                        outb[cc // 8, t, cc % 8, pl.ds(lst, _L)] = zeros
                    lvec = lst + iota
                    tvec = jnp.broadcast_to(t, (_L,))
                    plsc.store_scatter(
                        outb, [clo >> 3, tvec, clo & 7, lvec], vlo)
                    plsc.store_scatter(
                        outb, [chi >> 3, tvec, chi & 7, lvec], vhi)

                for tr in range(TRC):
                    pltpu.async_copy(
                        outb.at[tr], out_hbm.at[tr, pl.ds(t0, _CHT)],
                        sems[b][tr])

        for b in range(2):
            for tr in range(TRC):
                pltpu.make_async_copy(
                    out_hbm.at[tr, pl.ds(0, _CHT)], outbs[b].at[tr],
                    sems[b][tr]).wait()

    return encode


def kernel(atomic_numbers, zs):
    n = atomic_numbers.shape[0]
    C = zs.shape[0]
    TRC = -(-C // 8)
    NT = -(-n // 128)
    enc = _make_sc_encode(n, C)
    out4 = enc(atomic_numbers.astype(jnp.int32), zs.astype(jnp.int32))
    return out4.transpose(1, 3, 0, 2).reshape(NT * 128, TRC * 8)[:n, :C]
